# baseline (speedup 1.0000x reference)
import jax
import jax.numpy as jnp
from jax import lax
from jax.experimental import pallas as pl
from jax.experimental.pallas import tpu as pltpu
from jax.experimental.pallas import tpu_sc as plsc

N = 10000
E = 160000
D = 256
H = 256
O = 128

NS = 16
NPAD = 10016
RPT = NPAD // NS
TPT = N // NS
TRASH = N
QW = 64

CHUNK = 128
CHUNKS = 80
GCH = 20
GROUPS = CHUNKS // GCH
NBUF = 4
EPAD = NS * CHUNKS * CHUNK
EROWS = EPAD // CHUNK

STEPS_A = [(o, min(64, RPT - o)) for o in range(0, RPT, 64)]
STEPS_T = [(o, min(64, TPT - o)) for o in range(0, TPT, 64)]

BLK = 400
GRID = N // BLK


def _make_sc_agg(with_deg):
    mesh = plsc.VectorSubcoreMesh(core_axis_name="c", subcore_axis_name="s")

    out_type = [jax.ShapeDtypeStruct((4 * NPAD, QW), jnp.float32)]
    scratch = [
        pltpu.VMEM_SHARED((N, QW), jnp.float32),
        pltpu.VMEM_SHARED((NPAD, QW), jnp.float32),
        pltpu.VMEM((GCH, CHUNK), jnp.int32),
        pltpu.VMEM((GCH, CHUNK), jnp.int32),
    ] + [pltpu.VMEM((CHUNK, QW), jnp.float32) for _ in range(NBUF)] + [
        pltpu.SemaphoreType.DMA for _ in range(2 * NBUF + 1)
    ]
    if with_deg:
        out_type.append(jax.ShapeDtypeStruct((2 * NPAD, 16), jnp.float32))
        scratch += [
            pltpu.VMEM_SHARED((NPAD, 16), jnp.float32),
            pltpu.VMEM((CHUNK, 16), jnp.float32),
        ]

    def body(*refs):
        if with_deg:
            (table4, src2, dst2, z64, z16, ones16,
             out_agg, out_deg, tblsp, acc, srcv, dstv) = refs[:12]
            rbufs = refs[12:12 + NBUF]
            gsems = refs[12 + NBUF:12 + 2 * NBUF]
            ssems = refs[12 + 2 * NBUF:12 + 3 * NBUF]
            dsem = refs[12 + 3 * NBUF]
            dacc, dbuf = refs[13 + 3 * NBUF], refs[14 + 3 * NBUF]
        else:
            (table4, src2, dst2, z64,
             out_agg, tblsp, acc, srcv, dstv) = refs[:9]
            rbufs = refs[9:9 + NBUF]
            gsems = refs[9 + NBUF:9 + 2 * NBUF]
            ssems = refs[9 + 2 * NBUF:9 + 3 * NBUF]
            dsem = refs[9 + 3 * NBUF]

        cid = lax.axis_index("c")
        tid = lax.axis_index("s")
        r0 = tid * RPT
        t0 = tid * TPT

        def make_edge_group(deg_pass):
            def edge_group(g, carry):
                pltpu.sync_copy(
                    src2.at[pl.ds(tid * CHUNKS + g * GCH, GCH)], srcv)
                pltpu.sync_copy(
                    dst2.at[pl.ds(tid * CHUNKS + g * GCH, GCH)], dstv)
                gd = [None] * NBUF
                sd = [None] * NBUF
                dds = []
                for p in range(NBUF - 1):
                    gd[p] = pltpu.async_copy(tblsp.at[srcv.at[p]], rbufs[p],
                                             gsems[p])
                for j in range(GCH):
                    b = j % NBUF
                    gd[b].wait()
                    nj = j + NBUF - 1
                    if nj < GCH:
                        nb = nj % NBUF
                        if sd[nb] is not None:
                            sd[nb].wait()
                        gd[nb] = pltpu.async_copy(tblsp.at[srcv.at[nj]],
                                                  rbufs[nb], gsems[nb])
                    sd[b] = pltpu.async_copy(
                        rbufs[b], acc.at[dstv.at[j]], ssems[b], add=True)
                    if deg_pass:
                        dds.append(pltpu.async_copy(
                            dbuf, dacc.at[dstv.at[j]], dsem, add=True))
                for b in range(NBUF):
                    if sd[b] is not None:
                        sd[b].wait()
                for dd in dds:
                    dd.wait()
                return carry
            return edge_group

        for p in (0, 1):
            q = 2 * p + cid
            for off, sz in STEPS_T:
                pltpu.sync_copy(table4.at[pl.ds(q * N + t0 + off, sz)],
                                rbufs[0].at[pl.ds(0, sz)])
                pltpu.sync_copy(rbufs[0].at[pl.ds(0, sz)],
                                tblsp.at[pl.ds(t0 + off, sz)])
            pltpu.sync_copy(z64, rbufs[1])
            for off, sz in STEPS_A:
                pltpu.sync_copy(rbufs[1].at[pl.ds(0, sz)],
                                acc.at[pl.ds(r0 + off, sz)])
            if with_deg and p == 0:
                pltpu.sync_copy(z16, dbuf)
                for off, sz in STEPS_A:
                    pltpu.sync_copy(dbuf.at[pl.ds(0, sz)],
                                    dacc.at[pl.ds(r0 + off, sz)])
                pltpu.sync_copy(ones16, dbuf)
            plsc.subcore_barrier()

            lax.fori_loop(0, GROUPS, make_edge_group(with_deg and p == 0), 0)
            plsc.subcore_barrier()

            o0 = q * NPAD + r0
            for off, sz in STEPS_A:
                pltpu.sync_copy(acc.at[pl.ds(r0 + off, sz)],
                                rbufs[0].at[pl.ds(0, sz)])
                pltpu.sync_copy(rbufs[0].at[pl.ds(0, sz)],
                                out_agg.at[pl.ds(o0 + off, sz)])
            if with_deg and p == 0:
                d0 = cid * NPAD + r0
                for off, sz in STEPS_A:
                    pltpu.sync_copy(dacc.at[pl.ds(r0 + off, sz)],
                                    dbuf.at[pl.ds(0, sz)])
                    pltpu.sync_copy(dbuf.at[pl.ds(0, sz)],
                                    out_deg.at[pl.ds(d0 + off, sz)])

    return pl.kernel(body, out_type=out_type, mesh=mesh, scratch_types=scratch,
                     compiler_params=pltpu.CompilerParams(
                         use_tc_tiling_on_sc=False))


def _quarter_specs():
    return [pl.BlockSpec((1, BLK, QW), (lambda i, q=q: (q, i, 0)))
            for q in range(4)]


def _tc_layer0(x, agg, deg, ws, wn, b):
    def body(x_ref, a0, a1, a2, a3, deg_ref, ws_ref, wn_ref, b_ref, out_ref):
        rdeg = 1.0 / jnp.maximum(deg_ref[0][:, :1], 1.0)
        acc = jnp.dot(x_ref[...], ws_ref[...], preferred_element_type=jnp.float32)
        for q, aq in enumerate((a0, a1, a2, a3)):
            acc += jnp.dot(aq[0] * rdeg, wn_ref[q * QW:(q + 1) * QW, :],
                           preferred_element_type=jnp.float32)
        h = jnp.maximum(acc + b_ref[...], 0.0)
        for q in range(4):
            out_ref[q] = h[:, q * QW:(q + 1) * QW]

    return pl.pallas_call(
        body,
        grid=(GRID,),
        in_specs=[pl.BlockSpec((BLK, D), lambda i: (i, 0))]
        + _quarter_specs()
        + [
            pl.BlockSpec((1, BLK, 16), lambda i: (0, i, 0)),
            pl.BlockSpec((D, H), lambda i: (0, 0)),
            pl.BlockSpec((D, H), lambda i: (0, 0)),
            pl.BlockSpec((1, H), lambda i: (0, 0)),
        ],
        out_specs=pl.BlockSpec((4, BLK, QW), lambda i: (0, i, 0)),
        out_shape=jax.ShapeDtypeStruct((4, N, QW), jnp.float32),
    )(x, agg, agg, agg, agg, deg, ws, wn, b)


def _tc_layer1(h1s, agg, deg, ws, wn, wfc, b1, bfc):
    def body(h1_ref, a0, a1, a2, a3, deg_ref, ws_ref, wn_ref, wfc_ref,
             b1_ref, bfc_ref, out_ref):
        rdeg = 1.0 / jnp.maximum(deg_ref[0][:, :1], 1.0)
        acc = jnp.dot(h1_ref[0], ws_ref[:QW, :],
                      preferred_element_type=jnp.float32)
        for q in range(1, 4):
            acc += jnp.dot(h1_ref[q], ws_ref[q * QW:(q + 1) * QW, :],
                           preferred_element_type=jnp.float32)
        for q, aq in enumerate((a0, a1, a2, a3)):
            acc += jnp.dot(aq[0] * rdeg, wn_ref[q * QW:(q + 1) * QW, :],
                           preferred_element_type=jnp.float32)
        h2 = jnp.maximum(acc + b1_ref[...], 0.0)
        out_ref[...] = jnp.dot(h2, wfc_ref[...],
                               preferred_element_type=jnp.float32) + bfc_ref[...]

    return pl.pallas_call(
        body,
        grid=(GRID,),
        in_specs=[pl.BlockSpec((4, BLK, QW), lambda i: (0, i, 0))]
        + _quarter_specs()
        + [
            pl.BlockSpec((1, BLK, 16), lambda i: (0, i, 0)),
            pl.BlockSpec((H, H), lambda i: (0, 0)),
            pl.BlockSpec((H, H), lambda i: (0, 0)),
            pl.BlockSpec((H, O), lambda i: (0, 0)),
            pl.BlockSpec((1, H), lambda i: (0, 0)),
            pl.BlockSpec((1, O), lambda i: (0, 0)),
        ],
        out_specs=pl.BlockSpec((BLK, O), lambda i: (i, 0)),
        out_shape=jax.ShapeDtypeStruct((N, O), jnp.float32),
    )(h1s, agg, agg, agg, agg, deg, ws, wn, wfc, b1, bfc)


def _tc_gru(y0, y1, y2, wihT, whhT, bih, bhh):
    def body(y0_ref, y1_ref, y2_ref, wih_ref, whh_ref, bih_ref, bhh_ref, out_ref):
        h = jnp.zeros((BLK, H), jnp.float32)
        for y_ref in (y0_ref, y1_ref, y2_ref):
            gi = jnp.dot(y_ref[...], wih_ref[...],
                         preferred_element_type=jnp.float32) + bih_ref[...]
            gh = jnp.dot(h, whh_ref[...],
                         preferred_element_type=jnp.float32) + bhh_ref[...]
            r = jax.nn.sigmoid(gi[:, :H] + gh[:, :H])
            z = jax.nn.sigmoid(gi[:, H:2 * H] + gh[:, H:2 * H])
            n = jnp.tanh(gi[:, 2 * H:] + r * gh[:, 2 * H:])
            h = (1.0 - z) * n + z * h
        out_ref[...] = h

    return pl.pallas_call(
        body,
        grid=(GRID,),
        in_specs=[
            pl.BlockSpec((BLK, O), lambda i: (i, 0)),
            pl.BlockSpec((BLK, O), lambda i: (i, 0)),
            pl.BlockSpec((BLK, O), lambda i: (i, 0)),
            pl.BlockSpec((O, 3 * H), lambda i: (0, 0)),
            pl.BlockSpec((H, 3 * H), lambda i: (0, 0)),
            pl.BlockSpec((1, 3 * H), lambda i: (0, 0)),
            pl.BlockSpec((1, 3 * H), lambda i: (0, 0)),
        ],
        out_specs=pl.BlockSpec((BLK, H), lambda i: (i, 0)),
        out_shape=jax.ShapeDtypeStruct((N, H), jnp.float32),
    )(y0, y1, y2, wihT, whhT, bih, bhh)


def kernel(features_0, features_1, features_2,
           edge_index_0, edge_index_1, edge_index_2,
           W_self_0, W_neigh_0, b_0, W_self_1, W_neigh_1, b_1, W_fc, b_fc,
           W_ih, W_hh, b_ih, b_hh):
    sc_agg_deg = _make_sc_agg(True)
    sc_agg = _make_sc_agg(False)

    z64 = jnp.zeros((CHUNK, QW), jnp.float32)
    z16 = jnp.zeros((CHUNK, 16), jnp.float32)
    ones16 = jnp.ones((CHUNK, 16), jnp.float32)

    b0r = b_0.reshape(1, H)
    b1r = b_1.reshape(1, H)
    bfcr = b_fc.reshape(1, O)
    wihT = W_ih.T
    whhT = W_hh.T
    bihr = b_ih.reshape(1, 3 * H)
    bhhr = b_hh.reshape(1, 3 * H)

    ys = []
    for feats, ei in ((features_0, edge_index_0),
                      (features_1, edge_index_1),
                      (features_2, edge_index_2)):
        src = ei[0]
        dst = ei[1]
        src_p = jnp.concatenate([src, jnp.zeros((EPAD - E,), jnp.int32)])
        dst_p = jnp.concatenate([dst, jnp.full((EPAD - E,), TRASH, jnp.int32)])
        src2 = src_p.reshape(EROWS, CHUNK)
        dst2 = dst_p.reshape(EROWS, CHUNK)

        table0 = jnp.concatenate(
            [feats[:, q * QW:(q + 1) * QW] for q in range(4)], axis=0)
        agg0, deg = sc_agg_deg(table0, src2, dst2, z64, z16, ones16)
        agg0 = agg0.reshape(4, NPAD, QW)
        deg = deg.reshape(2, NPAD, 16)
        h1s = _tc_layer0(feats, agg0, deg, W_self_0, W_neigh_0, b0r)
        (agg1,) = sc_agg(h1s.reshape(4 * N, QW), src2, dst2, z64)
        agg1 = agg1.reshape(4, NPAD, QW)
        y = _tc_layer1(h1s, agg1, deg, W_self_1, W_neigh_1, W_fc, b1r, bfcr)
        ys.append(y)

    final = _tc_gru(ys[0], ys[1], ys[2], wihT, whhT, bihr, bhhr)
    yearly = jnp.stack(ys, axis=1)
    return final, yearly


# speedup vs baseline: 1.0579x; 1.0579x over previous
import jax
import jax.numpy as jnp
from jax import lax
from jax.experimental import pallas as pl
from jax.experimental.pallas import tpu as pltpu
from jax.experimental.pallas import tpu_sc as plsc

N = 10000
E = 160000
D = 256
H = 256
O = 128

NS = 16
NPAD = 10016
RPT = NPAD // NS
TPT = N // NS
TRASH = N
QW = 64

CHUNK = 64
CHUNKS = 160
GCH = 32
GROUPS = CHUNKS // GCH
NBUF = 4
EPAD = NS * CHUNKS * CHUNK
EROWS = EPAD // CHUNK

STEPS_A = [(o, min(64, RPT - o)) for o in range(0, RPT, 64)]
STEPS_T = [(o, min(64, TPT - o)) for o in range(0, TPT, 64)]

BLK = 400
GRID = N // BLK


def _make_sc_agg(with_deg):
    mesh = plsc.VectorSubcoreMesh(core_axis_name="c", subcore_axis_name="s")

    out_type = [jax.ShapeDtypeStruct((4 * NPAD, QW), jnp.float32)]
    scratch = [
        pltpu.VMEM_SHARED((N, QW), jnp.float32),
        pltpu.VMEM_SHARED((NPAD, QW), jnp.float32),
        pltpu.VMEM((GCH, CHUNK), jnp.int32),
        pltpu.VMEM((GCH, CHUNK), jnp.int32),
    ] + [pltpu.VMEM((CHUNK, QW), jnp.float32) for _ in range(NBUF)] + [
        pltpu.SemaphoreType.DMA for _ in range(2 * NBUF + 1)
    ]
    if with_deg:
        out_type.append(jax.ShapeDtypeStruct((2 * NPAD, 16), jnp.float32))
        scratch += [
            pltpu.VMEM_SHARED((NPAD, 16), jnp.float32),
            pltpu.VMEM((CHUNK, 16), jnp.float32),
        ]

    def body(*refs):
        if with_deg:
            (table4, src2, dst2, z64, z16, ones16,
             out_agg, out_deg, tblsp, acc, srcv, dstv) = refs[:12]
            rbufs = refs[12:12 + NBUF]
            gsems = refs[12 + NBUF:12 + 2 * NBUF]
            ssems = refs[12 + 2 * NBUF:12 + 3 * NBUF]
            dsem = refs[12 + 3 * NBUF]
            dacc, dbuf = refs[13 + 3 * NBUF], refs[14 + 3 * NBUF]
        else:
            (table4, src2, dst2, z64,
             out_agg, tblsp, acc, srcv, dstv) = refs[:9]
            rbufs = refs[9:9 + NBUF]
            gsems = refs[9 + NBUF:9 + 2 * NBUF]
            ssems = refs[9 + 2 * NBUF:9 + 3 * NBUF]
            dsem = refs[9 + 3 * NBUF]

        cid = lax.axis_index("c")
        tid = lax.axis_index("s")
        r0 = tid * RPT
        t0 = tid * TPT

        def make_edge_group(deg_pass):
            def edge_group(g, carry):
                pltpu.sync_copy(
                    src2.at[pl.ds(tid * CHUNKS + g * GCH, GCH)], srcv)
                pltpu.sync_copy(
                    dst2.at[pl.ds(tid * CHUNKS + g * GCH, GCH)], dstv)
                gd = [None] * NBUF
                sd = [None] * NBUF
                dds = []
                for p in range(NBUF - 1):
                    gd[p] = pltpu.async_copy(tblsp.at[srcv.at[p]], rbufs[p],
                                             gsems[p])
                for j in range(GCH):
                    b = j % NBUF
                    gd[b].wait()
                    nj = j + NBUF - 1
                    if nj < GCH:
                        nb = nj % NBUF
                        if sd[nb] is not None:
                            sd[nb].wait()
                        gd[nb] = pltpu.async_copy(tblsp.at[srcv.at[nj]],
                                                  rbufs[nb], gsems[nb])
                    sd[b] = pltpu.async_copy(
                        rbufs[b], acc.at[dstv.at[j]], ssems[b], add=True)
                    if deg_pass:
                        dds.append(pltpu.async_copy(
                            dbuf, dacc.at[dstv.at[j]], dsem, add=True))
                for b in range(NBUF):
                    if sd[b] is not None:
                        sd[b].wait()
                for dd in dds:
                    dd.wait()
                return carry
            return edge_group

        for p in (0, 1):
            q = 2 * p + cid
            for off, sz in STEPS_T:
                pltpu.sync_copy(table4.at[pl.ds(q * N + t0 + off, sz)],
                                rbufs[0].at[pl.ds(0, sz)])
                pltpu.sync_copy(rbufs[0].at[pl.ds(0, sz)],
                                tblsp.at[pl.ds(t0 + off, sz)])
            pltpu.sync_copy(z64, rbufs[1])
            for off, sz in STEPS_A:
                pltpu.sync_copy(rbufs[1].at[pl.ds(0, sz)],
                                acc.at[pl.ds(r0 + off, sz)])
            if with_deg and p == 0:
                pltpu.sync_copy(z16, dbuf)
                for off, sz in STEPS_A:
                    pltpu.sync_copy(dbuf.at[pl.ds(0, sz)],
                                    dacc.at[pl.ds(r0 + off, sz)])
                pltpu.sync_copy(ones16, dbuf)
            plsc.subcore_barrier()

            lax.fori_loop(0, GROUPS, make_edge_group(with_deg and p == 0), 0)
            plsc.subcore_barrier()

            o0 = q * NPAD + r0
            for off, sz in STEPS_A:
                pltpu.sync_copy(acc.at[pl.ds(r0 + off, sz)],
                                rbufs[0].at[pl.ds(0, sz)])
                pltpu.sync_copy(rbufs[0].at[pl.ds(0, sz)],
                                out_agg.at[pl.ds(o0 + off, sz)])
            if with_deg and p == 0:
                d0 = cid * NPAD + r0
                for off, sz in STEPS_A:
                    pltpu.sync_copy(dacc.at[pl.ds(r0 + off, sz)],
                                    dbuf.at[pl.ds(0, sz)])
                    pltpu.sync_copy(dbuf.at[pl.ds(0, sz)],
                                    out_deg.at[pl.ds(d0 + off, sz)])

    return pl.kernel(body, out_type=out_type, mesh=mesh, scratch_types=scratch,
                     compiler_params=pltpu.CompilerParams(
                         use_tc_tiling_on_sc=False))


def _quarter_specs():
    return [pl.BlockSpec((1, BLK, QW), (lambda i, q=q: (q, i, 0)))
            for q in range(4)]


def _tc_layer0(x, agg, deg, ws, wn, b):
    def body(x_ref, a0, a1, a2, a3, deg_ref, ws_ref, wn_ref, b_ref, out_ref):
        rdeg = 1.0 / jnp.maximum(deg_ref[0][:, :1], 1.0)
        acc = jnp.dot(x_ref[...], ws_ref[...], preferred_element_type=jnp.float32)
        for q, aq in enumerate((a0, a1, a2, a3)):
            acc += jnp.dot(aq[0] * rdeg, wn_ref[q * QW:(q + 1) * QW, :],
                           preferred_element_type=jnp.float32)
        h = jnp.maximum(acc + b_ref[...], 0.0)
        for q in range(4):
            out_ref[q] = h[:, q * QW:(q + 1) * QW]

    return pl.pallas_call(
        body,
        grid=(GRID,),
        in_specs=[pl.BlockSpec((BLK, D), lambda i: (i, 0))]
        + _quarter_specs()
        + [
            pl.BlockSpec((1, BLK, 16), lambda i: (0, i, 0)),
            pl.BlockSpec((D, H), lambda i: (0, 0)),
            pl.BlockSpec((D, H), lambda i: (0, 0)),
            pl.BlockSpec((1, H), lambda i: (0, 0)),
        ],
        out_specs=pl.BlockSpec((4, BLK, QW), lambda i: (0, i, 0)),
        out_shape=jax.ShapeDtypeStruct((4, N, QW), jnp.float32),
    )(x, agg, agg, agg, agg, deg, ws, wn, b)


def _tc_layer1(h1s, agg, deg, ws, wn, wfc, b1, bfc):
    def body(h1_ref, a0, a1, a2, a3, deg_ref, ws_ref, wn_ref, wfc_ref,
             b1_ref, bfc_ref, out_ref):
        rdeg = 1.0 / jnp.maximum(deg_ref[0][:, :1], 1.0)
        acc = jnp.dot(h1_ref[0], ws_ref[:QW, :],
                      preferred_element_type=jnp.float32)
        for q in range(1, 4):
            acc += jnp.dot(h1_ref[q], ws_ref[q * QW:(q + 1) * QW, :],
                           preferred_element_type=jnp.float32)
        for q, aq in enumerate((a0, a1, a2, a3)):
            acc += jnp.dot(aq[0] * rdeg, wn_ref[q * QW:(q + 1) * QW, :],
                           preferred_element_type=jnp.float32)
        h2 = jnp.maximum(acc + b1_ref[...], 0.0)
        out_ref[...] = jnp.dot(h2, wfc_ref[...],
                               preferred_element_type=jnp.float32) + bfc_ref[...]

    return pl.pallas_call(
        body,
        grid=(GRID,),
        in_specs=[pl.BlockSpec((4, BLK, QW), lambda i: (0, i, 0))]
        + _quarter_specs()
        + [
            pl.BlockSpec((1, BLK, 16), lambda i: (0, i, 0)),
            pl.BlockSpec((H, H), lambda i: (0, 0)),
            pl.BlockSpec((H, H), lambda i: (0, 0)),
            pl.BlockSpec((H, O), lambda i: (0, 0)),
            pl.BlockSpec((1, H), lambda i: (0, 0)),
            pl.BlockSpec((1, O), lambda i: (0, 0)),
        ],
        out_specs=pl.BlockSpec((BLK, O), lambda i: (i, 0)),
        out_shape=jax.ShapeDtypeStruct((N, O), jnp.float32),
    )(h1s, agg, agg, agg, agg, deg, ws, wn, wfc, b1, bfc)


def _tc_gru(y0, y1, y2, wihT, whhT, bih, bhh):
    def body(y0_ref, y1_ref, y2_ref, wih_ref, whh_ref, bih_ref, bhh_ref, out_ref):
        h = jnp.zeros((BLK, H), jnp.float32)
        for y_ref in (y0_ref, y1_ref, y2_ref):
            gi = jnp.dot(y_ref[...], wih_ref[...],
                         preferred_element_type=jnp.float32) + bih_ref[...]
            gh = jnp.dot(h, whh_ref[...],
                         preferred_element_type=jnp.float32) + bhh_ref[...]
            r = jax.nn.sigmoid(gi[:, :H] + gh[:, :H])
            z = jax.nn.sigmoid(gi[:, H:2 * H] + gh[:, H:2 * H])
            n = jnp.tanh(gi[:, 2 * H:] + r * gh[:, 2 * H:])
            h = (1.0 - z) * n + z * h
        out_ref[...] = h

    return pl.pallas_call(
        body,
        grid=(GRID,),
        in_specs=[
            pl.BlockSpec((BLK, O), lambda i: (i, 0)),
            pl.BlockSpec((BLK, O), lambda i: (i, 0)),
            pl.BlockSpec((BLK, O), lambda i: (i, 0)),
            pl.BlockSpec((O, 3 * H), lambda i: (0, 0)),
            pl.BlockSpec((H, 3 * H), lambda i: (0, 0)),
            pl.BlockSpec((1, 3 * H), lambda i: (0, 0)),
            pl.BlockSpec((1, 3 * H), lambda i: (0, 0)),
        ],
        out_specs=pl.BlockSpec((BLK, H), lambda i: (i, 0)),
        out_shape=jax.ShapeDtypeStruct((N, H), jnp.float32),
    )(y0, y1, y2, wihT, whhT, bih, bhh)


def kernel(features_0, features_1, features_2,
           edge_index_0, edge_index_1, edge_index_2,
           W_self_0, W_neigh_0, b_0, W_self_1, W_neigh_1, b_1, W_fc, b_fc,
           W_ih, W_hh, b_ih, b_hh):
    sc_agg_deg = _make_sc_agg(True)
    sc_agg = _make_sc_agg(False)

    z64 = jnp.zeros((CHUNK, QW), jnp.float32)
    z16 = jnp.zeros((CHUNK, 16), jnp.float32)
    ones16 = jnp.ones((CHUNK, 16), jnp.float32)

    b0r = b_0.reshape(1, H)
    b1r = b_1.reshape(1, H)
    bfcr = b_fc.reshape(1, O)
    wihT = W_ih.T
    whhT = W_hh.T
    bihr = b_ih.reshape(1, 3 * H)
    bhhr = b_hh.reshape(1, 3 * H)

    ys = []
    for feats, ei in ((features_0, edge_index_0),
                      (features_1, edge_index_1),
                      (features_2, edge_index_2)):
        src = ei[0]
        dst = ei[1]
        src_p = jnp.concatenate([src, jnp.zeros((EPAD - E,), jnp.int32)])
        dst_p = jnp.concatenate([dst, jnp.full((EPAD - E,), TRASH, jnp.int32)])
        src2 = src_p.reshape(EROWS, CHUNK)
        dst2 = dst_p.reshape(EROWS, CHUNK)

        table0 = jnp.concatenate(
            [feats[:, q * QW:(q + 1) * QW] for q in range(4)], axis=0)
        agg0, deg = sc_agg_deg(table0, src2, dst2, z64, z16, ones16)
        agg0 = agg0.reshape(4, NPAD, QW)
        deg = deg.reshape(2, NPAD, 16)
        h1s = _tc_layer0(feats, agg0, deg, W_self_0, W_neigh_0, b0r)
        (agg1,) = sc_agg(h1s.reshape(4 * N, QW), src2, dst2, z64)
        agg1 = agg1.reshape(4, NPAD, QW)
        y = _tc_layer1(h1s, agg1, deg, W_self_1, W_neigh_1, W_fc, b1r, bfcr)
        ys.append(y)

    final = _tc_gru(ys[0], ys[1], ys[2], wihT, whhT, bihr, bhhr)
    yearly = jnp.stack(ys, axis=1)
    return final, yearly


# speedup vs baseline: 1.1373x; 1.0750x over previous
import jax
import jax.numpy as jnp
from jax import lax
from jax.experimental import pallas as pl
from jax.experimental.pallas import tpu as pltpu
from jax.experimental.pallas import tpu_sc as plsc

N = 10000
E = 160000
D = 256
H = 256
O = 128

NS = 16
NPAD = 10016
RPT = NPAD // NS
TPT = N // NS
TRASH = N
QW = 64

CHUNK = 64
CHUNKS = 160
GCH = 32
GROUPS = CHUNKS // GCH
NBUF = 4
EPAD = NS * CHUNKS * CHUNK
EROWS = EPAD // CHUNK

STEPS_A = [(o, min(64, RPT - o)) for o in range(0, RPT, 64)]
STEPS_T = [(o, min(64, TPT - o)) for o in range(0, TPT, 64)]

BLK = 400
GRID = N // BLK


def _make_sc_agg(with_deg):
    mesh = plsc.VectorSubcoreMesh(core_axis_name="c", subcore_axis_name="s")

    out_type = [jax.ShapeDtypeStruct((4 * NPAD, QW), jnp.float32)]
    scratch = [
        pltpu.VMEM_SHARED((N, QW), jnp.float32),
        pltpu.VMEM_SHARED((NPAD, QW), jnp.float32),
        pltpu.VMEM((GCH, CHUNK), jnp.int32),
        pltpu.VMEM((GCH, CHUNK), jnp.int32),
    ] + [pltpu.VMEM((CHUNK, QW), jnp.float32) for _ in range(NBUF)] + [
        pltpu.SemaphoreType.DMA for _ in range(2 * NBUF + 1)
    ]
    if with_deg:
        out_type.append(jax.ShapeDtypeStruct((2 * NPAD, 16), jnp.float32))
        scratch += [
            pltpu.VMEM_SHARED((NPAD, 16), jnp.float32),
            pltpu.VMEM((CHUNK, 16), jnp.float32),
        ]

    def body(*refs):
        if with_deg:
            (table4, src2, dst2, zacc, zdeg, ones16,
             out_agg, out_deg, tblsp, acc, srcv, dstv) = refs[:12]
            rbufs = refs[12:12 + NBUF]
            gsems = refs[12 + NBUF:12 + 2 * NBUF]
            ssems = refs[12 + 2 * NBUF:12 + 3 * NBUF]
            dsem = refs[12 + 3 * NBUF]
            dacc, dbuf = refs[13 + 3 * NBUF], refs[14 + 3 * NBUF]
        else:
            (table4, src2, dst2, zacc,
             out_agg, tblsp, acc, srcv, dstv) = refs[:9]
            rbufs = refs[9:9 + NBUF]
            gsems = refs[9 + NBUF:9 + 2 * NBUF]
            ssems = refs[9 + 2 * NBUF:9 + 3 * NBUF]
            dsem = refs[9 + 3 * NBUF]

        cid = lax.axis_index("c")
        tid = lax.axis_index("s")
        r0 = tid * RPT
        t0 = tid * TPT

        def make_edge_group(deg_pass):
            def edge_group(g, carry):
                pltpu.sync_copy(
                    src2.at[pl.ds(tid * CHUNKS + g * GCH, GCH)], srcv)
                pltpu.sync_copy(
                    dst2.at[pl.ds(tid * CHUNKS + g * GCH, GCH)], dstv)
                gd = [None] * NBUF
                sd = [None] * NBUF
                dds = []
                for p in range(NBUF - 1):
                    gd[p] = pltpu.async_copy(tblsp.at[srcv.at[p]], rbufs[p],
                                             gsems[p])
                for j in range(GCH):
                    b = j % NBUF
                    gd[b].wait()
                    nj = j + NBUF - 1
                    if nj < GCH:
                        nb = nj % NBUF
                        if sd[nb] is not None:
                            sd[nb].wait()
                        gd[nb] = pltpu.async_copy(tblsp.at[srcv.at[nj]],
                                                  rbufs[nb], gsems[nb])
                    sd[b] = pltpu.async_copy(
                        rbufs[b], acc.at[dstv.at[j]], ssems[b], add=True)
                    if deg_pass:
                        dds.append(pltpu.async_copy(
                            dbuf, dacc.at[dstv.at[j]], dsem, add=True))
                for b in range(NBUF):
                    if sd[b] is not None:
                        sd[b].wait()
                for dd in dds:
                    dd.wait()
                return carry
            return edge_group

        for p in (0, 1):
            q = 2 * p + cid
            pltpu.sync_copy(table4.at[pl.ds(q * N + t0, TPT)],
                            tblsp.at[pl.ds(t0, TPT)])
            pltpu.sync_copy(zacc, acc.at[pl.ds(r0, RPT)])
            if with_deg and p == 0:
                pltpu.sync_copy(zdeg, dacc.at[pl.ds(r0, RPT)])
                pltpu.sync_copy(ones16, dbuf)
            plsc.subcore_barrier()

            lax.fori_loop(0, GROUPS, make_edge_group(with_deg and p == 0), 0)
            plsc.subcore_barrier()

            o0 = q * NPAD + r0
            pltpu.sync_copy(acc.at[pl.ds(r0, RPT)],
                            out_agg.at[pl.ds(o0, RPT)])
            if with_deg and p == 0:
                d0 = cid * NPAD + r0
                pltpu.sync_copy(dacc.at[pl.ds(r0, RPT)],
                                out_deg.at[pl.ds(d0, RPT)])

    return pl.kernel(body, out_type=out_type, mesh=mesh, scratch_types=scratch,
                     compiler_params=pltpu.CompilerParams(
                         use_tc_tiling_on_sc=False))


def _quarter_specs():
    return [pl.BlockSpec((1, BLK, QW), (lambda i, q=q: (q, i, 0)))
            for q in range(4)]


def _tc_layer0(x, agg, deg, ws, wn, b):
    def body(x_ref, a0, a1, a2, a3, deg_ref, ws_ref, wn_ref, b_ref, out_ref):
        rdeg = 1.0 / jnp.maximum(deg_ref[0][:, :1], 1.0)
        acc = jnp.dot(x_ref[...], ws_ref[...], preferred_element_type=jnp.float32)
        for q, aq in enumerate((a0, a1, a2, a3)):
            acc += jnp.dot(aq[0] * rdeg, wn_ref[q * QW:(q + 1) * QW, :],
                           preferred_element_type=jnp.float32)
        h = jnp.maximum(acc + b_ref[...], 0.0)
        for q in range(4):
            out_ref[q] = h[:, q * QW:(q + 1) * QW]

    return pl.pallas_call(
        body,
        grid=(GRID,),
        in_specs=[pl.BlockSpec((BLK, D), lambda i: (i, 0))]
        + _quarter_specs()
        + [
            pl.BlockSpec((1, BLK, 16), lambda i: (0, i, 0)),
            pl.BlockSpec((D, H), lambda i: (0, 0)),
            pl.BlockSpec((D, H), lambda i: (0, 0)),
            pl.BlockSpec((1, H), lambda i: (0, 0)),
        ],
        out_specs=pl.BlockSpec((4, BLK, QW), lambda i: (0, i, 0)),
        out_shape=jax.ShapeDtypeStruct((4, N, QW), jnp.float32),
    )(x, agg, agg, agg, agg, deg, ws, wn, b)


def _tc_layer1(h1s, agg, deg, ws, wn, wfc, b1, bfc):
    def body(h1_ref, a0, a1, a2, a3, deg_ref, ws_ref, wn_ref, wfc_ref,
             b1_ref, bfc_ref, out_ref):
        rdeg = 1.0 / jnp.maximum(deg_ref[0][:, :1], 1.0)
        acc = jnp.dot(h1_ref[0], ws_ref[:QW, :],
                      preferred_element_type=jnp.float32)
        for q in range(1, 4):
            acc += jnp.dot(h1_ref[q], ws_ref[q * QW:(q + 1) * QW, :],
                           preferred_element_type=jnp.float32)
        for q, aq in enumerate((a0, a1, a2, a3)):
            acc += jnp.dot(aq[0] * rdeg, wn_ref[q * QW:(q + 1) * QW, :],
                           preferred_element_type=jnp.float32)
        h2 = jnp.maximum(acc + b1_ref[...], 0.0)
        out_ref[...] = jnp.dot(h2, wfc_ref[...],
                               preferred_element_type=jnp.float32) + bfc_ref[...]

    return pl.pallas_call(
        body,
        grid=(GRID,),
        in_specs=[pl.BlockSpec((4, BLK, QW), lambda i: (0, i, 0))]
        + _quarter_specs()
        + [
            pl.BlockSpec((1, BLK, 16), lambda i: (0, i, 0)),
            pl.BlockSpec((H, H), lambda i: (0, 0)),
            pl.BlockSpec((H, H), lambda i: (0, 0)),
            pl.BlockSpec((H, O), lambda i: (0, 0)),
            pl.BlockSpec((1, H), lambda i: (0, 0)),
            pl.BlockSpec((1, O), lambda i: (0, 0)),
        ],
        out_specs=pl.BlockSpec((BLK, O), lambda i: (i, 0)),
        out_shape=jax.ShapeDtypeStruct((N, O), jnp.float32),
    )(h1s, agg, agg, agg, agg, deg, ws, wn, wfc, b1, bfc)


def _tc_gru(y0, y1, y2, wihT, whhT, bih, bhh):
    def body(y0_ref, y1_ref, y2_ref, wih_ref, whh_ref, bih_ref, bhh_ref, out_ref):
        h = jnp.zeros((BLK, H), jnp.float32)
        for y_ref in (y0_ref, y1_ref, y2_ref):
            gi = jnp.dot(y_ref[...], wih_ref[...],
                         preferred_element_type=jnp.float32) + bih_ref[...]
            gh = jnp.dot(h, whh_ref[...],
                         preferred_element_type=jnp.float32) + bhh_ref[...]
            r = jax.nn.sigmoid(gi[:, :H] + gh[:, :H])
            z = jax.nn.sigmoid(gi[:, H:2 * H] + gh[:, H:2 * H])
            n = jnp.tanh(gi[:, 2 * H:] + r * gh[:, 2 * H:])
            h = (1.0 - z) * n + z * h
        out_ref[...] = h

    return pl.pallas_call(
        body,
        grid=(GRID,),
        in_specs=[
            pl.BlockSpec((BLK, O), lambda i: (i, 0)),
            pl.BlockSpec((BLK, O), lambda i: (i, 0)),
            pl.BlockSpec((BLK, O), lambda i: (i, 0)),
            pl.BlockSpec((O, 3 * H), lambda i: (0, 0)),
            pl.BlockSpec((H, 3 * H), lambda i: (0, 0)),
            pl.BlockSpec((1, 3 * H), lambda i: (0, 0)),
            pl.BlockSpec((1, 3 * H), lambda i: (0, 0)),
        ],
        out_specs=pl.BlockSpec((BLK, H), lambda i: (i, 0)),
        out_shape=jax.ShapeDtypeStruct((N, H), jnp.float32),
    )(y0, y1, y2, wihT, whhT, bih, bhh)


def kernel(features_0, features_1, features_2,
           edge_index_0, edge_index_1, edge_index_2,
           W_self_0, W_neigh_0, b_0, W_self_1, W_neigh_1, b_1, W_fc, b_fc,
           W_ih, W_hh, b_ih, b_hh):
    sc_agg_deg = _make_sc_agg(True)
    sc_agg = _make_sc_agg(False)

    zacc = jnp.zeros((RPT, QW), jnp.float32)
    zdeg = jnp.zeros((RPT, 16), jnp.float32)
    ones16 = jnp.ones((CHUNK, 16), jnp.float32)

    b0r = b_0.reshape(1, H)
    b1r = b_1.reshape(1, H)
    bfcr = b_fc.reshape(1, O)
    wihT = W_ih.T
    whhT = W_hh.T
    bihr = b_ih.reshape(1, 3 * H)
    bhhr = b_hh.reshape(1, 3 * H)

    ys = []
    for feats, ei in ((features_0, edge_index_0),
                      (features_1, edge_index_1),
                      (features_2, edge_index_2)):
        src = ei[0]
        dst = ei[1]
        src_p = jnp.concatenate([src, jnp.zeros((EPAD - E,), jnp.int32)])
        dst_p = jnp.concatenate([dst, jnp.full((EPAD - E,), TRASH, jnp.int32)])
        src2 = src_p.reshape(EROWS, CHUNK)
        dst2 = dst_p.reshape(EROWS, CHUNK)

        table0 = jnp.concatenate(
            [feats[:, q * QW:(q + 1) * QW] for q in range(4)], axis=0)
        agg0, deg = sc_agg_deg(table0, src2, dst2, zacc, zdeg, ones16)
        agg0 = agg0.reshape(4, NPAD, QW)
        deg = deg.reshape(2, NPAD, 16)
        h1s = _tc_layer0(feats, agg0, deg, W_self_0, W_neigh_0, b0r)
        (agg1,) = sc_agg(h1s.reshape(4 * N, QW), src2, dst2, zacc)
        agg1 = agg1.reshape(4, NPAD, QW)
        y = _tc_layer1(h1s, agg1, deg, W_self_1, W_neigh_1, W_fc, b1r, bfcr)
        ys.append(y)

    final = _tc_gru(ys[0], ys[1], ys[2], wihT, whhT, bihr, bhhr)
    yearly = jnp.stack(ys, axis=1)
    return final, yearly
